# baseline (speedup 1.0000x reference)
import functools

import jax
import jax.numpy as jnp
import numpy as np
from jax import lax
from jax.experimental import pallas as pl
from jax.experimental.pallas import tpu as pltpu
from jax.experimental.pallas import tpu_sc as plsc

N = 131072
P = 2
N_LEVELS = 16
F = 2
LOG2_T = 17
T = 2 ** LOG2_T
TG = T // 8
P2 = np.uint32(2654435761).astype(np.int32)
P3 = np.uint32(805459861).astype(np.int32)
RES = [float(np.floor(16.0 * (1.5 ** l))) for l in range(N_LEVELS)]

NC = 2
NS = 16
NW = NC * NS
PTS = N // NW
C = 256
NCHUNK = PTS // C
G = C // 16


def _body(xs_hbm, ys_hbm, zs_hbm, grans_hbm, out_hbm,
          cx, cy, cz, eb, idx0, idx1, col0, col1, wb0, wb1, rows0, rows1,
          outb, sem0, sem1):
    wid = lax.axis_index("s") * NC + lax.axis_index("c")
    base = wid * PTS
    iota = jnp.arange(16, dtype=jnp.int32)
    idxb = (idx0, idx1)
    colb = (col0, col1)
    wbb = (wb0, wb1)
    rowsb = (rows0, rows1)
    semb = (sem0, sem1)

    def chunk_body(kc, _):
        cb = base + kc * C
        pltpu.sync_copy(xs_hbm.at[pl.ds(cb, C)], cx)
        pltpu.sync_copy(ys_hbm.at[pl.ds(cb, C)], cy)
        pltpu.sync_copy(zs_hbm.at[pl.ds(cb, C)], cz)

        def prep(g, _):
            s = g * 16
            x = cx[pl.ds(s, 16)]
            y = cy[pl.ds(s, 16)]
            z = cz[pl.ds(s, 16)]
            ex = jnp.clip((x * 2.0).astype(jnp.int32), 0, P - 1)
            ey = jnp.clip((y * 2.0).astype(jnp.int32), 0, P - 1)
            ez = jnp.clip((z * 2.0).astype(jnp.int32), 0, P - 1)
            eb[pl.ds(s, 16)] = (ex * 4 + ey * 2 + ez) * (N_LEVELS * TG)
            return 0

        lax.fori_loop(0, G, prep, 0)

        def hash_level(l, buf):
            res = jnp.float32(RES[l])
            ib = idxb[buf]
            cob = colb[buf]
            wb = wbb[buf]

            def hash_grp(g, _):
                s = g * 16
                x = cx[pl.ds(s, 16)]
                y = cy[pl.ds(s, 16)]
                z = cz[pl.ds(s, 16)]
                g0 = eb[pl.ds(s, 16)] + (l * TG)
                sx = x * res
                sy = y * res
                sz = z * res
                ix = sx.astype(jnp.int32)
                iy = sy.astype(jnp.int32)
                iz = sz.astype(jnp.int32)
                fx = sx - ix.astype(jnp.float32)
                fy = sy - iy.astype(jnp.float32)
                fz = sz - iz.astype(jnp.float32)
                hx = (ix, ix + 1)
                hy = (iy * P2, (iy + 1) * P2)
                hz = (iz * P3, (iz + 1) * P3)
                wx = (1.0 - fx, fx)
                wy = (1.0 - fy, fy)
                wz = (1.0 - fz, fz)
                wxy = {(i, j): wx[i] * wy[j] for i in (0, 1) for j in (0, 1)}
                for i in (0, 1):
                    for j in (0, 1):
                        for k in (0, 1):
                            c = i * 4 + j * 2 + k
                            h = (hx[i] ^ hy[j] ^ hz[k]) & (T - 1)
                            ib[pl.ds(c * C + s, 16)] = g0 + (h >> 3)
                            cob[pl.ds(c * C + s, 16)] = (h & 7) * 2
                            wb[pl.ds(c * C + s, 16)] = wxy[(i, j)] * wz[k]
                return 0

            lax.fori_loop(0, G, hash_grp, 0)

        def acc_level(l, buf):
            rows = rowsb[buf]
            cob = colb[buf]
            wb = wbb[buf]
            half = iota // 2
            feat = iota & 1
            outq = half * (2 * N_LEVELS) + feat + (2 * l)

            def acc_grp(g, _):
                s8 = g * 8
                acc = jnp.zeros((16,), jnp.float32)
                for c in range(8):
                    pidx = half + (c * C + s8)
                    colv = plsc.load_gather(cob, [pidx]) + feat
                    v = plsc.load_gather(rows, [pidx, colv])
                    wpair = plsc.load_gather(wb, [pidx])
                    acc = acc + v * wpair
                plsc.store_scatter(outb, [outq + s8 * (2 * N_LEVELS)], acc)
                return 0

            lax.fori_loop(0, 2 * G, acc_grp, 0)

        hash_level(0, 0)
        dma = pltpu.async_copy(grans_hbm.at[idx0], rows0, sem0)
        for l in range(1, N_LEVELS):
            b = l & 1
            pb = 1 - b
            hash_level(l, b)
            dma_next = pltpu.async_copy(grans_hbm.at[idxb[b]], rowsb[b], semb[b])
            dma.wait()
            acc_level(l - 1, pb)
            dma = dma_next
        dma.wait()
        acc_level(N_LEVELS - 1, (N_LEVELS - 1) & 1)

        pltpu.sync_copy(outb, out_hbm.at[pl.ds(cb * (2 * N_LEVELS), C * 2 * N_LEVELS)])
        return 0

    lax.fori_loop(0, NCHUNK, chunk_body, 0)


def kernel(coords, tables):
    c32 = coords.astype(jnp.float32)
    xs, ys, zs = c32[:, 0], c32[:, 1], c32[:, 2]
    grans = tables.reshape(P ** 3 * N_LEVELS * TG, 16)
    mesh = plsc.VectorSubcoreMesh(core_axis_name="c", subcore_axis_name="s")
    run = pl.kernel(
        _body,
        out_type=jax.ShapeDtypeStruct((N * N_LEVELS * F,), jnp.float32),
        mesh=mesh,
        scratch_types=[
            pltpu.VMEM((C,), jnp.float32),
            pltpu.VMEM((C,), jnp.float32),
            pltpu.VMEM((C,), jnp.float32),
            pltpu.VMEM((C,), jnp.int32),
            pltpu.VMEM((8 * C,), jnp.int32),
            pltpu.VMEM((8 * C,), jnp.int32),
            pltpu.VMEM((8 * C,), jnp.int32),
            pltpu.VMEM((8 * C,), jnp.int32),
            pltpu.VMEM((8 * C,), jnp.float32),
            pltpu.VMEM((8 * C,), jnp.float32),
            pltpu.VMEM((8 * C, 16), jnp.float32),
            pltpu.VMEM((8 * C, 16), jnp.float32),
            pltpu.VMEM((C * N_LEVELS * F,), jnp.float32),
            pltpu.SemaphoreType.DMA,
            pltpu.SemaphoreType.DMA,
        ],
        compiler_params=pltpu.CompilerParams(
            needs_layout_passes=False, use_tc_tiling_on_sc=False),
    )
    return run(xs, ys, zs, grans).reshape(N, N_LEVELS * F)


# speedup vs baseline: 21.1733x; 21.1733x over previous
import functools

import jax
import jax.numpy as jnp
import numpy as np
from jax import lax
from jax.experimental import pallas as pl
from jax.experimental.pallas import tpu as pltpu
from jax.experimental.pallas import tpu_sc as plsc

N = 131072
P = 2
N_LEVELS = 16
F = 2
LOG2_T = 17
T = 2 ** LOG2_T
TG = T // 8
P2 = np.uint32(2654435761).astype(np.int32)
P3 = np.uint32(805459861).astype(np.int32)
RES = [float(np.floor(16.0 * (1.5 ** l))) for l in range(N_LEVELS)]

NC = 2
NS = 16
NW = NC * NS
PTS = N // NW
C = 256
NCHUNK = PTS // C
G = C // 16


def _body(xs_hbm, ys_hbm, zs_hbm, grans_hbm, out_hbm,
          cx, cy, cz, eb, idx0, idx1, col0, col1, wb0, wb1, rows0, rows1,
          outb, sem0, sem1):
    wid = lax.axis_index("s") * NC + lax.axis_index("c")
    base = wid * PTS
    iota = jnp.arange(16, dtype=jnp.int32)
    idxb = (idx0, idx1)
    colb = (col0, col1)
    wbb = (wb0, wb1)
    rowsb = (rows0, rows1)
    semb = (sem0, sem1)

    def chunk_body(kc, _):
        cb = base + kc * C
        pltpu.sync_copy(xs_hbm.at[pl.ds(cb, C)], cx)
        pltpu.sync_copy(ys_hbm.at[pl.ds(cb, C)], cy)
        pltpu.sync_copy(zs_hbm.at[pl.ds(cb, C)], cz)

        def prep(g, _):
            s = g * 16
            x = cx[pl.ds(s, 16)]
            y = cy[pl.ds(s, 16)]
            z = cz[pl.ds(s, 16)]
            ex = jnp.clip((x * 2.0).astype(jnp.int32), 0, P - 1)
            ey = jnp.clip((y * 2.0).astype(jnp.int32), 0, P - 1)
            ez = jnp.clip((z * 2.0).astype(jnp.int32), 0, P - 1)
            eb[pl.ds(s, 16)] = (ex * 4 + ey * 2 + ez) * (N_LEVELS * TG)
            return 0

        lax.fori_loop(0, G, prep, 0)

        def hash_level(l, buf):
            res = jnp.float32(RES[l])
            ib = idxb[buf]
            cob = colb[buf]
            wb = wbb[buf]

            def hash_grp(g, _):
                s = g * 16
                x = cx[pl.ds(s, 16)]
                y = cy[pl.ds(s, 16)]
                z = cz[pl.ds(s, 16)]
                g0 = eb[pl.ds(s, 16)] + (l * TG)
                sx = x * res
                sy = y * res
                sz = z * res
                ix = sx.astype(jnp.int32)
                iy = sy.astype(jnp.int32)
                iz = sz.astype(jnp.int32)
                fx = sx - ix.astype(jnp.float32)
                fy = sy - iy.astype(jnp.float32)
                fz = sz - iz.astype(jnp.float32)
                hx = (ix, ix + 1)
                hy = (iy * P2, (iy + 1) * P2)
                hz = (iz * P3, (iz + 1) * P3)
                wx = (1.0 - fx, fx)
                wy = (1.0 - fy, fy)
                wz = (1.0 - fz, fz)
                wxy = {(i, j): wx[i] * wy[j] for i in (0, 1) for j in (0, 1)}
                for i in (0, 1):
                    for j in (0, 1):
                        for k in (0, 1):
                            c = i * 4 + j * 2 + k
                            h = (hx[i] ^ hy[j] ^ hz[k]) & (T - 1)
                            ib[pl.ds(c * C + s, 16)] = g0 + (h >> 3)
                            cob[pl.ds(c * C + s, 16)] = (h & 7) * 2
                            wb[pl.ds(c * C + s, 16)] = wxy[(i, j)] * wz[k]
                return 0

            lax.fori_loop(0, G, hash_grp, 0)

        def acc_level(l, buf):
            rows = rowsb[buf]
            cob = colb[buf]
            wb = wbb[buf]
            half = iota // 2
            feat = iota & 1
            outq = half * (2 * N_LEVELS) + feat + (2 * l)

            def acc_grp(g, _):
                s8 = g * 8
                acc = jnp.zeros((16,), jnp.float32)
                for c in range(8):
                    pidx = half + (c * C + s8)
                    colv = plsc.load_gather(cob, [pidx]) + feat
                    v = plsc.load_gather(rows, [pidx, colv])
                    wpair = plsc.load_gather(wb, [pidx])
                    acc = acc + v * wpair
                plsc.store_scatter(outb, [outq + s8 * (2 * N_LEVELS)], acc)
                return 0

            lax.fori_loop(0, 2 * G, acc_grp, 0)

        hash_level(0, 0)
        dma = pltpu.async_copy(grans_hbm.at[idx0], rows0, sem0)
        for l in range(1, N_LEVELS):
            b = l & 1
            pb = 1 - b
            hash_level(l, b)
            dma_next = pltpu.async_copy(grans_hbm.at[idxb[b]], rowsb[b], semb[b])
            dma.wait()
            acc_level(l - 1, pb)
            dma = dma_next
        dma.wait()
        acc_level(N_LEVELS - 1, (N_LEVELS - 1) & 1)

        pltpu.sync_copy(outb, out_hbm.at[pl.ds(cb * (2 * N_LEVELS), C * 2 * N_LEVELS)])
        return 0

    lax.fori_loop(0, NCHUNK, chunk_body, 0)


def _interleave_tc(x_ref, p_ref, o_ref):
    o_ref[...] = jnp.dot(x_ref[...], p_ref[...],
                         preferred_element_type=jnp.float32)


def kernel(coords, tables):
    c32 = coords.astype(jnp.float32)
    xs, ys, zs = c32[:, 0], c32[:, 1], c32[:, 2]
    phys = (tables.reshape(P ** 3, N_LEVELS, T // 128, 128, F)
            .transpose(0, 1, 2, 4, 3)
            .reshape(P ** 3 * N_LEVELS * T // 128, 2 * 128))
    perm = np.zeros((256, 256), dtype=np.float32)
    s = np.arange(128)
    perm[s, 2 * s] = 1.0
    perm[s + 128, 2 * s + 1] = 1.0
    n_rows = phys.shape[0]
    blk = 2048
    inter = pl.pallas_call(
        _interleave_tc,
        grid=(n_rows // blk,),
        in_specs=[
            pl.BlockSpec((blk, 256), lambda i: (i, 0)),
            pl.BlockSpec((256, 256), lambda i: (0, 0)),
        ],
        out_specs=pl.BlockSpec((blk, 256), lambda i: (i, 0)),
        out_shape=jax.ShapeDtypeStruct((n_rows, 256), jnp.float32),
    )(phys, jnp.asarray(perm))
    grans = inter.reshape(P ** 3 * N_LEVELS * TG, 16)
    mesh = plsc.VectorSubcoreMesh(core_axis_name="c", subcore_axis_name="s")
    run = pl.kernel(
        _body,
        out_type=jax.ShapeDtypeStruct((N * N_LEVELS * F,), jnp.float32),
        mesh=mesh,
        scratch_types=[
            pltpu.VMEM((C,), jnp.float32),
            pltpu.VMEM((C,), jnp.float32),
            pltpu.VMEM((C,), jnp.float32),
            pltpu.VMEM((C,), jnp.int32),
            pltpu.VMEM((8 * C,), jnp.int32),
            pltpu.VMEM((8 * C,), jnp.int32),
            pltpu.VMEM((8 * C,), jnp.int32),
            pltpu.VMEM((8 * C,), jnp.int32),
            pltpu.VMEM((8 * C,), jnp.float32),
            pltpu.VMEM((8 * C,), jnp.float32),
            pltpu.VMEM((8 * C, 16), jnp.float32),
            pltpu.VMEM((8 * C, 16), jnp.float32),
            pltpu.VMEM((C * N_LEVELS * F,), jnp.float32),
            pltpu.SemaphoreType.DMA,
            pltpu.SemaphoreType.DMA,
        ],
        compiler_params=pltpu.CompilerParams(
            needs_layout_passes=False, use_tc_tiling_on_sc=False),
    )
    return run(xs, ys, zs, grans).reshape(N, N_LEVELS * F)


# speedup vs baseline: 23.1933x; 1.0954x over previous
import functools

import jax
import jax.numpy as jnp
import numpy as np
from jax import lax
from jax.experimental import pallas as pl
from jax.experimental.pallas import tpu as pltpu
from jax.experimental.pallas import tpu_sc as plsc

N = 131072
P = 2
N_LEVELS = 16
F = 2
LOG2_T = 17
T = 2 ** LOG2_T
TG = T // 8
P2 = np.uint32(2654435761).astype(np.int32)
P3 = np.uint32(805459861).astype(np.int32)
RES = [float(np.floor(16.0 * (1.5 ** l))) for l in range(N_LEVELS)]

NC = 2
NS = 16
NW = NC * NS
PTS = N // NW
C = 256
NCHUNK = PTS // C
G = C // 16


def _body(xs_hbm, ys_hbm, zs_hbm, grans_hbm, out_hbm,
          cx, cy, cz, eb, idx0, idx1, col0, col1, wb0, wb1, rows0, rows1,
          outb, sem0, sem1):
    wid = lax.axis_index("s") * NC + lax.axis_index("c")
    base = wid * PTS
    iota = jnp.arange(16, dtype=jnp.int32)
    idxb = (idx0, idx1)
    colb = (col0, col1)
    wbb = (wb0, wb1)
    rowsb = (rows0, rows1)
    semb = (sem0, sem1)

    def chunk_body(kc, _):
        cb = base + kc * C
        pltpu.sync_copy(xs_hbm.at[pl.ds(cb, C)], cx)
        pltpu.sync_copy(ys_hbm.at[pl.ds(cb, C)], cy)
        pltpu.sync_copy(zs_hbm.at[pl.ds(cb, C)], cz)

        def prep(g, _):
            s = g * 16
            x = cx[pl.ds(s, 16)]
            y = cy[pl.ds(s, 16)]
            z = cz[pl.ds(s, 16)]
            ex = jnp.clip((x * 2.0).astype(jnp.int32), 0, P - 1)
            ey = jnp.clip((y * 2.0).astype(jnp.int32), 0, P - 1)
            ez = jnp.clip((z * 2.0).astype(jnp.int32), 0, P - 1)
            eb[pl.ds(s, 16)] = (ex * 4 + ey * 2 + ez) * (N_LEVELS * TG)
            return 0

        lax.fori_loop(0, G, prep, 0)

        def hash_level(l, buf):
            res = jnp.float32(RES[l])
            ib = idxb[buf]
            cob = colb[buf]
            wb = wbb[buf]

            def hash_grp(g, _):
                s = g * 16
                x = cx[pl.ds(s, 16)]
                y = cy[pl.ds(s, 16)]
                z = cz[pl.ds(s, 16)]
                g0 = eb[pl.ds(s, 16)] + (l * TG)
                sx = x * res
                sy = y * res
                sz = z * res
                ix = sx.astype(jnp.int32)
                iy = sy.astype(jnp.int32)
                iz = sz.astype(jnp.int32)
                fx = sx - ix.astype(jnp.float32)
                fy = sy - iy.astype(jnp.float32)
                fz = sz - iz.astype(jnp.float32)
                hx = (ix, ix + 1)
                hy = (iy * P2, (iy + 1) * P2)
                hz = (iz * P3, (iz + 1) * P3)
                wx = (1.0 - fx, fx)
                wy = (1.0 - fy, fy)
                wz = (1.0 - fz, fz)
                wxy = {(i, j): wx[i] * wy[j] for i in (0, 1) for j in (0, 1)}
                for i in (0, 1):
                    for j in (0, 1):
                        for k in (0, 1):
                            c = i * 4 + j * 2 + k
                            h = (hx[i] ^ hy[j] ^ hz[k]) & (T - 1)
                            ib[pl.ds(c * C + s, 16)] = g0 + (h >> 3)
                            cob[pl.ds(c * C + s, 16)] = (h & 7) * 2
                            wb[pl.ds(c * C + s, 16)] = wxy[(i, j)] * wz[k]
                return 0

            lax.fori_loop(0, G, hash_grp, 0)

        def acc_level(l, buf):
            rows = rowsb[buf]
            cob = colb[buf]
            wb = wbb[buf]
            half = iota // 2
            feat = iota & 1
            outq = half * (2 * N_LEVELS) + feat + (2 * l)

            def acc_grp(g, _):
                s8 = g * 8
                acc = jnp.zeros((16,), jnp.float32)
                for c in range(8):
                    pidx = half + (c * C + s8)
                    colv = plsc.load_gather(cob, [pidx]) + feat
                    v = plsc.load_gather(rows, [pidx, colv])
                    wpair = plsc.load_gather(wb, [pidx])
                    acc = acc + v * wpair
                plsc.store_scatter(outb, [outq + s8 * (2 * N_LEVELS)], acc)
                return 0

            lax.fori_loop(0, 2 * G, acc_grp, 0)

        hash_level(0, 0)
        dma = pltpu.async_copy(grans_hbm.at[idx0], rows0, sem0)
        for l in range(1, N_LEVELS):
            b = l & 1
            pb = 1 - b
            hash_level(l, b)
            dma_next = pltpu.async_copy(grans_hbm.at[idxb[b]], rowsb[b], semb[b])
            dma.wait()
            acc_level(l - 1, pb)
            dma = dma_next
        dma.wait()
        acc_level(N_LEVELS - 1, (N_LEVELS - 1) & 1)

        pltpu.sync_copy(outb, out_hbm.at[pl.ds(cb * (2 * N_LEVELS), C * 2 * N_LEVELS)])
        return 0

    lax.fori_loop(0, NCHUNK, chunk_body, 0)


def _interleave_tc(x_ref, p0_ref, p1_ref, o_ref):
    f0 = x_ref[:, 0, :]
    f1 = x_ref[:, 1, :]
    o_ref[...] = (jnp.dot(f0, p0_ref[...], preferred_element_type=jnp.float32)
                  + jnp.dot(f1, p1_ref[...], preferred_element_type=jnp.float32))


def kernel(coords, tables):
    c32 = coords.astype(jnp.float32)
    xs, ys, zs = c32[:, 0], c32[:, 1], c32[:, 2]
    phys = (tables.reshape(P ** 3, N_LEVELS, T // 128, 128, F)
            .transpose(0, 1, 2, 4, 3)
            .reshape(P ** 3 * N_LEVELS * T // 128, 2, 128))
    s = np.arange(128)
    p0 = np.zeros((128, 256), dtype=np.float32)
    p0[s, 2 * s] = 1.0
    p1 = np.zeros((128, 256), dtype=np.float32)
    p1[s, 2 * s + 1] = 1.0
    n_rows = phys.shape[0]
    blk = 2048
    inter = pl.pallas_call(
        _interleave_tc,
        grid=(n_rows // blk,),
        in_specs=[
            pl.BlockSpec((blk, 2, 128), lambda i: (i, 0, 0)),
            pl.BlockSpec((128, 256), lambda i: (0, 0)),
            pl.BlockSpec((128, 256), lambda i: (0, 0)),
        ],
        out_specs=pl.BlockSpec((blk, 256), lambda i: (i, 0)),
        out_shape=jax.ShapeDtypeStruct((n_rows, 256), jnp.float32),
    )(phys, jnp.asarray(p0), jnp.asarray(p1))
    grans = inter.reshape(P ** 3 * N_LEVELS * TG, 16)
    mesh = plsc.VectorSubcoreMesh(core_axis_name="c", subcore_axis_name="s")
    run = pl.kernel(
        _body,
        out_type=jax.ShapeDtypeStruct((N * N_LEVELS * F,), jnp.float32),
        mesh=mesh,
        scratch_types=[
            pltpu.VMEM((C,), jnp.float32),
            pltpu.VMEM((C,), jnp.float32),
            pltpu.VMEM((C,), jnp.float32),
            pltpu.VMEM((C,), jnp.int32),
            pltpu.VMEM((8 * C,), jnp.int32),
            pltpu.VMEM((8 * C,), jnp.int32),
            pltpu.VMEM((8 * C,), jnp.int32),
            pltpu.VMEM((8 * C,), jnp.int32),
            pltpu.VMEM((8 * C,), jnp.float32),
            pltpu.VMEM((8 * C,), jnp.float32),
            pltpu.VMEM((8 * C, 16), jnp.float32),
            pltpu.VMEM((8 * C, 16), jnp.float32),
            pltpu.VMEM((C * N_LEVELS * F,), jnp.float32),
            pltpu.SemaphoreType.DMA,
            pltpu.SemaphoreType.DMA,
        ],
        compiler_params=pltpu.CompilerParams(
            needs_layout_passes=False, use_tc_tiling_on_sc=False),
    )
    return run(xs, ys, zs, grans).reshape(N, N_LEVELS * F)
